# baseline (speedup 1.0000x reference)
import jax
import jax.numpy as jnp
from jax import lax
from jax.experimental import pallas as pl
from jax.experimental.pallas import tpu as pltpu
from jax.experimental.pallas import tpu_sc as plsc

_MAXLEN = 8192
_EMBED = 128
_BATCH = 4
_NC = 2
_NS = 16
_ROWS = _MAXLEN // (_NC * _NS)
_CHUNK = 128
_HALVES = _ROWS // _CHUNK
_NCHUNKS = _BATCH * _HALVES
_RING = 4


def _sc_body(x_hbm, pos_hbm, idx_hbm, out_hbm,
             idx0, idx1, ring0, ring1, ring2, ring3, *sems):
    wid = lax.axis_index("s") * _NC + lax.axis_index("c")
    t0 = wid * _ROWS

    ring = (ring0, ring1, ring2, ring3)
    lsems = sems[0:_RING]
    asems = sems[_RING:2 * _RING]
    ssems = sems[2 * _RING:3 * _RING]
    idx = (idx0, idx1)

    pltpu.sync_copy(idx_hbm.at[_HALVES * wid], idx0)
    pltpu.sync_copy(idx_hbm.at[_HALVES * wid + 1], idx1)

    loads, adds, stores = {}, {}, {}
    for step in range(_NCHUNKS + 2):
        c = step
        if c < _NCHUNKS:
            if c >= _RING:
                stores[c - _RING].wait()
            b, h = divmod(c, _HALVES)
            rows = pl.ds(t0 + h * _CHUNK, _CHUNK)
            loads[c] = pltpu.async_copy(
                x_hbm.at[b, rows], ring[c % _RING], lsems[c % _RING])
        c = step - 1
        if 0 <= c < _NCHUNKS:
            loads[c].wait()
            adds[c] = pltpu.async_copy(
                pos_hbm.at[idx[c % _HALVES]], ring[c % _RING],
                asems[c % _RING], add=True)
        c = step - 2
        if 0 <= c < _NCHUNKS:
            adds[c].wait()
            b, h = divmod(c, _HALVES)
            rows = pl.ds(t0 + h * _CHUNK, _CHUNK)
            stores[c] = pltpu.async_copy(
                ring[c % _RING], out_hbm.at[b, rows], ssems[c % _RING])
    for c in range(_NCHUNKS - _RING, _NCHUNKS):
        stores[c].wait()


def kernel(x, pos_table):
    idx = jnp.arange(_MAXLEN, dtype=jnp.int32).reshape(-1, _CHUNK)
    mesh = plsc.VectorSubcoreMesh(core_axis_name="c", subcore_axis_name="s",
                                  num_cores=_NC, num_subcores=_NS)
    run = pl.kernel(
        _sc_body,
        out_type=jax.ShapeDtypeStruct((_BATCH, _MAXLEN, _EMBED), jnp.float32),
        mesh=mesh,
        scratch_types=(
            [pltpu.VMEM((_CHUNK,), jnp.int32)] * 2
            + [pltpu.VMEM((_CHUNK, _EMBED), jnp.float32)] * _RING
            + [pltpu.SemaphoreType.DMA] * (3 * _RING)
        ),
    )
    return run(x, pos_table, idx)
